# baseline (speedup 1.0000x reference)
import functools

import jax
import jax.numpy as jnp
from jax import lax
from jax.experimental import pallas as pl
from jax.experimental.pallas import tpu as pltpu
from jax.experimental.pallas import tpu_sc as plsc

V = 1000000
E = 64
H = 4
C = 3
B = 1024
T = 50
EPS = 1e-05

NC = 2
NS = 16
NW = NC * NS
BT = B * T
RPW = BT // NW
CH = 80
NCH = RPW // CH
HB = NCH // 2


TBS = 32768
TQB = TBS // 4
TGRID = (V + TBS - 1) // TBS
VP = TGRID * TQB


def _bf16_hi_lo(lo, hi):
    lo16 = lax.bitcast_convert_type(
        lo.astype(jnp.bfloat16), jnp.uint16).astype(jnp.uint32)
    hi16 = lax.bitcast_convert_type(
        hi.astype(jnp.bfloat16), jnp.uint16).astype(jnp.uint32)
    return (hi16 << 16) | lo16


def _tr_body(x_ref, o_ref):
    x = x_ref[...]
    t1 = _bf16_hi_lo(x[:, :TQB], x[:, TQB:2 * TQB])
    t2 = _bf16_hi_lo(x[:, 2 * TQB:3 * TQB], x[:, 3 * TQB:])
    o_ref[...] = jnp.concatenate([t1, t2], axis=0).T


def _transpose_pack(table_t):
    return pl.pallas_call(
        _tr_body,
        grid=(TGRID,),
        in_specs=[pl.BlockSpec((E, TBS), lambda i: (0, i))],
        out_specs=pl.BlockSpec((TQB, 2 * E), lambda i: (i, 0)),
        out_shape=jax.ShapeDtypeStruct((VP, 2 * E), jnp.uint32),
    )(table_t)


def _sc_gather(idx3, table2):
    mesh = plsc.VectorSubcoreMesh(core_axis_name="c", subcore_axis_name="s")

    @functools.partial(
        pl.kernel,
        mesh=mesh,
        out_type=jax.ShapeDtypeStruct((BT, 2 * E), jnp.uint32),
        scratch_types=[
            pltpu.VMEM((NCH, CH), jnp.int32),
            pltpu.VMEM((HB * CH, 2 * E), jnp.uint32),
            pltpu.SemaphoreType.DMA,
        ],
        compiler_params=pltpu.CompilerParams(use_tc_tiling_on_sc=False),
    )
    def k(idx_hbm, table_hbm, out_hbm, idx_v, rows_v, sem):
        wid = lax.axis_index("s") * NC + lax.axis_index("c")
        pltpu.sync_copy(idx_hbm.at[wid], idx_v)
        for h in range(2):
            copies = []
            for j in range(HB):
                copies.append(
                    pltpu.async_copy(
                        table_hbm.at[idx_v.at[h * HB + j]],
                        rows_v.at[pl.ds(j * CH, CH)],
                        sem,
                    )
                )
            for cp in copies:
                cp.wait()
            pltpu.sync_copy(
                rows_v, out_hbm.at[pl.ds(wid * RPW + h * HB * CH, HB * CH)]
            )

    return k(idx3, table2)


TB = 2
BLK = TB * B
TAIL = 2 * B


def _tc_body(g_ref, gp_ref, par_ref, parp_ref, m6_ref, wt_ref, b_ref,
             lnw_ref, lnb_ref, o_ref):
    i = pl.program_id(0)
    g2 = g_ref[...]
    gp2 = gp_ref[...]
    par = par_ref[...]
    parp = parp_ref[...]

    def unpack(quad, q):
        w = jnp.where(q >= 2, quad[:, E:], quad[:, :E])
        bits = jnp.where((q & 1) == 1, w & jnp.uint32(0xFFFF0000), w << 16)
        return lax.bitcast_convert_type(bits, jnp.float32)

    gc = unpack(g2, par)
    gp = unpack(gp2, parp)
    gp = jnp.where(i == 0, 0.0, gp)
    full = jnp.concatenate([gp, gc], axis=0)
    d3 = jnp.dot(full, m6_ref[...], preferred_element_type=jnp.float32)
    s2 = d3[TAIL:, 2:3] * gc
    s1 = d3[B:B + BLK, 1:2] * full[B:B + BLK]
    s0 = d3[:BLK, 0:1] * full[:BLK]
    out = (s2 + s1 + s0) * (1.0 / (H * C))
    y = jnp.dot(out, wt_ref[...], preferred_element_type=jnp.float32)
    y = y + b_ref[...]
    mean = jnp.mean(y, axis=-1, keepdims=True)
    yc = y - mean
    var = jnp.mean(yc * yc, axis=-1, keepdims=True)
    yn = yc * lax.rsqrt(var + EPS) * lnw_ref[...] + lnb_ref[...]
    o = yn * jax.nn.sigmoid(yn)
    for tl in range(TB):
        o_ref[tl] = o[tl * B:(tl + 1) * B].T


def _tc_compute(g2, par, m6, wt, bias, lnw, lnb):
    grid = BT // BLK
    return pl.pallas_call(
        _tc_body,
        grid=(grid,),
        in_specs=[
            pl.BlockSpec((BLK, 2 * E), lambda i: (i, 0)),
            pl.BlockSpec((TAIL, 2 * E),
                         lambda i: (jnp.maximum(i * (BLK // TAIL) - 1, 0), 0)),
            pl.BlockSpec((BLK, 1), lambda i: (i, 0)),
            pl.BlockSpec((TAIL, 1),
                         lambda i: (jnp.maximum(i * (BLK // TAIL) - 1, 0), 0)),
            pl.BlockSpec((E, 2 * E), lambda i: (0, 0)),
            pl.BlockSpec((E, E), lambda i: (0, 0)),
            pl.BlockSpec((1, E), lambda i: (0, 0)),
            pl.BlockSpec((1, E), lambda i: (0, 0)),
            pl.BlockSpec((1, E), lambda i: (0, 0)),
        ],
        out_specs=pl.BlockSpec((TB, E, B), lambda i: (i, 0, 0)),
        out_shape=jax.ShapeDtypeStruct((T, E, B), jnp.float32),
    )(g2, g2, par, par, m6, wt, bias, lnw, lnb)


def kernel(input, embed, pos_embed_weight, ffn_w, ffn_b, ln_w, ln_b):
    idx = input.astype(jnp.int32).T.reshape(-1)
    row = (idx // TBS) * TQB + (idx & (TQB - 1))
    idx3 = row.reshape(NW, NCH, CH)
    par = ((idx // TQB) & 3).reshape(BT, 1)
    table2 = _transpose_pack(embed.T)
    g2 = _sc_gather(idx3, table2)
    m = pos_embed_weight.reshape(H, E, C).sum(axis=0)
    m6 = jnp.concatenate([m, jnp.zeros((E, 2 * E - C), m.dtype)], axis=1)
    out = _tc_compute(
        g2,
        par,
        m6,
        ffn_w.T,
        ffn_b.reshape(1, E),
        ln_w.reshape(1, E),
        ln_b.reshape(1, E),
    )
    return out.transpose(2, 0, 1)


# speedup vs baseline: 1.0876x; 1.0876x over previous
import functools

import jax
import jax.numpy as jnp
from jax import lax
from jax.experimental import pallas as pl
from jax.experimental.pallas import tpu as pltpu
from jax.experimental.pallas import tpu_sc as plsc

V = 1000000
E = 64
H = 4
C = 3
B = 1024
T = 50
EPS = 1e-05

NC = 2
NS = 16
NW = NC * NS
BT = B * T
RPW = BT // NW
CH = 80
NCH = RPW // CH
HB = NCH // 2


TBS = 32768
TQB = TBS // 4
TGRID = (V + TBS - 1) // TBS
VP = TGRID * TQB


def _bf16_hi_lo(lo, hi):
    lo16 = lax.bitcast_convert_type(
        lo.astype(jnp.bfloat16), jnp.uint16).astype(jnp.uint32)
    hi16 = lax.bitcast_convert_type(
        hi.astype(jnp.bfloat16), jnp.uint16).astype(jnp.uint32)
    return (hi16 << 16) | lo16


def _tr_body(x_ref, o_ref):
    x = x_ref[...]
    t1 = _bf16_hi_lo(x[:, :TQB], x[:, TQB:2 * TQB])
    t2 = _bf16_hi_lo(x[:, 2 * TQB:3 * TQB], x[:, 3 * TQB:])
    o_ref[...] = jnp.concatenate([t1, t2], axis=0).T


def _transpose_pack(table_t):
    return pl.pallas_call(
        _tr_body,
        grid=(TGRID,),
        in_specs=[pl.BlockSpec((E, TBS), lambda i: (0, i))],
        out_specs=pl.BlockSpec((TQB, 2 * E), lambda i: (i, 0)),
        out_shape=jax.ShapeDtypeStruct((VP, 2 * E), jnp.uint32),
    )(table_t)


def _sc_gather(idx3, table2):
    mesh = plsc.VectorSubcoreMesh(core_axis_name="c", subcore_axis_name="s")

    @functools.partial(
        pl.kernel,
        mesh=mesh,
        out_type=jax.ShapeDtypeStruct((BT, 2 * E), jnp.uint32),
        scratch_types=[
            pltpu.VMEM((NCH, CH), jnp.int32),
            pltpu.VMEM((HB * CH, 2 * E), jnp.uint32),
            pltpu.SemaphoreType.DMA,
        ],
        compiler_params=pltpu.CompilerParams(use_tc_tiling_on_sc=False),
    )
    def k(idx_hbm, table_hbm, out_hbm, idx_v, rows_v, sem):
        wid = lax.axis_index("s") * NC + lax.axis_index("c")
        pltpu.sync_copy(idx_hbm.at[wid], idx_v)
        for h in range(2):
            copies = []
            for j in range(HB):
                copies.append(
                    pltpu.async_copy(
                        table_hbm.at[idx_v.at[h * HB + j]],
                        rows_v.at[pl.ds(j * CH, CH)],
                        sem,
                    )
                )
            for cp in copies:
                cp.wait()
            pltpu.sync_copy(
                rows_v, out_hbm.at[pl.ds(wid * RPW + h * HB * CH, HB * CH)]
            )

    return k(idx3, table2)


TB = 10
BLK = TB * B
TAIL = 2 * B


def _tc_body(g_ref, gp_ref, par_ref, parp_ref, m6_ref, wt_ref, b_ref,
             lnw_ref, lnb_ref, o_ref):
    i = pl.program_id(0)
    g2 = g_ref[...]
    gp2 = gp_ref[...]
    par = par_ref[...]
    parp = parp_ref[...]

    def unpack(quad, q):
        w = jnp.where(q >= 2, quad[:, E:], quad[:, :E])
        bits = jnp.where((q & 1) == 1, w & jnp.uint32(0xFFFF0000), w << 16)
        return lax.bitcast_convert_type(bits, jnp.float32)

    gc = unpack(g2, par)
    gp = unpack(gp2, parp)
    gp = jnp.where(i == 0, 0.0, gp)
    full = jnp.concatenate([gp, gc], axis=0)
    d3 = jnp.dot(full, m6_ref[...], preferred_element_type=jnp.float32)
    s2 = d3[TAIL:, 2:3] * gc
    s1 = d3[B:B + BLK, 1:2] * full[B:B + BLK]
    s0 = d3[:BLK, 0:1] * full[:BLK]
    out = (s2 + s1 + s0) * (1.0 / (H * C))
    y = jnp.dot(out, wt_ref[...], preferred_element_type=jnp.float32)
    y = y + b_ref[...]
    mean = jnp.mean(y, axis=-1, keepdims=True)
    yc = y - mean
    var = jnp.mean(yc * yc, axis=-1, keepdims=True)
    yn = yc * lax.rsqrt(var + EPS) * lnw_ref[...] + lnb_ref[...]
    o = yn * jax.nn.sigmoid(yn)
    for tl in range(TB):
        o_ref[tl] = o[tl * B:(tl + 1) * B].T


def _tc_compute(g2, par, m6, wt, bias, lnw, lnb):
    grid = BT // BLK
    return pl.pallas_call(
        _tc_body,
        grid=(grid,),
        in_specs=[
            pl.BlockSpec((BLK, 2 * E), lambda i: (i, 0)),
            pl.BlockSpec((TAIL, 2 * E),
                         lambda i: (jnp.maximum(i * (BLK // TAIL) - 1, 0), 0)),
            pl.BlockSpec((BLK, 1), lambda i: (i, 0)),
            pl.BlockSpec((TAIL, 1),
                         lambda i: (jnp.maximum(i * (BLK // TAIL) - 1, 0), 0)),
            pl.BlockSpec((E, 2 * E), lambda i: (0, 0)),
            pl.BlockSpec((E, E), lambda i: (0, 0)),
            pl.BlockSpec((1, E), lambda i: (0, 0)),
            pl.BlockSpec((1, E), lambda i: (0, 0)),
            pl.BlockSpec((1, E), lambda i: (0, 0)),
        ],
        out_specs=pl.BlockSpec((TB, E, B), lambda i: (i, 0, 0)),
        out_shape=jax.ShapeDtypeStruct((T, E, B), jnp.float32),
    )(g2, g2, par, par, m6, wt, bias, lnw, lnb)


def kernel(input, embed, pos_embed_weight, ffn_w, ffn_b, ln_w, ln_b):
    idx = input.astype(jnp.int32).T.reshape(-1)
    row = (idx // TBS) * TQB + (idx & (TQB - 1))
    idx3 = row.reshape(NW, NCH, CH)
    par = ((idx // TQB) & 3).reshape(BT, 1)
    table2 = _transpose_pack(embed.T)
    g2 = _sc_gather(idx3, table2)
    m = pos_embed_weight.reshape(H, E, C).sum(axis=0)
    m6 = jnp.concatenate([m, jnp.zeros((E, 2 * E - C), m.dtype)], axis=1)
    out = _tc_compute(
        g2,
        par,
        m6,
        ffn_w.T,
        ffn_b.reshape(1, E),
        ln_w.reshape(1, E),
        ln_b.reshape(1, E),
    )
    return out.transpose(2, 0, 1)
